# Initial kernel scaffold; baseline (speedup 1.0000x reference)
import jax
import jax.numpy as jnp
from jax.experimental import pallas as pl


def kernel(x, l0_wbig, l0_bsp, l0_wt, l0_bt, l1_wbig, l1_bsp, l1_wt, l1_bt, l2_wbig, l2_bsp, l2_wt, l2_bt, l2_wres, l2_bres):
    raise NotImplementedError("write your pallas kernel here")


# speedup vs baseline: 1.3285x; 1.3285x over previous
import jax
import jax.numpy as jnp
from jax.experimental import pallas as pl
from jax.experimental.pallas import tpu as pltpu

_BF = jnp.bfloat16

_CP = pltpu.CompilerParams(
    dimension_semantics=("parallel",),
    vmem_limit_bytes=64 * 1024 * 1024,
)


def _spatial_body(x_ref, w_ref, b_ref, o_ref):
    y = jnp.dot(x_ref[...], w_ref[...], preferred_element_type=jnp.float32)
    o_ref[...] = jnp.maximum(y + b_ref[...], 0.0).astype(o_ref.dtype)


def _spatial(x2d, w, b, rows):
    M, K = x2d.shape
    Nout = w.shape[1]
    return pl.pallas_call(
        _spatial_body,
        out_shape=jax.ShapeDtypeStruct((M, Nout), _BF),
        grid=(M // rows,),
        in_specs=[pl.BlockSpec((rows, K), lambda g: (g, 0)),
                  pl.BlockSpec((K, Nout), lambda g: (0, 0)),
                  pl.BlockSpec((1, Nout), lambda g: (0, 0))],
        out_specs=pl.BlockSpec((rows, Nout), lambda g: (g, 0)),
        compiler_params=_CP,
    )(x2d, w, b)


def _make_temporal64_body(res_kind, *, SB, T, V, C):
    TV = T * V
    PAD = 4 * V

    def _body(*refs):
        if res_kind == "zero":
            y_ref, wg0_ref, wg1_ref, w8_ref, bt_ref, o_ref = refs
        else:
            y_ref, r_ref, wg0_ref, wg1_ref, w8_ref, bt_ref, o_ref = refs
        z = jnp.zeros((PAD, C), _BF)
        for s in range(SB):
            ys = y_ref[s * TV:(s + 1) * TV, :]
            yp = jnp.concatenate([z, ys, z], axis=0)
            q = jnp.concatenate(
                [yp[0:TV + 128], yp[32:TV + 160],
                 yp[64:TV + 192], yp[96:TV + 224]], axis=1)
            acc = jnp.dot(q[0:TV], wg0_ref[...],
                          preferred_element_type=jnp.float32)
            acc = acc + jnp.dot(q[128:TV + 128], wg1_ref[...],
                                preferred_element_type=jnp.float32)
            acc = acc + jnp.dot(yp[256:TV + 256], w8_ref[...],
                                preferred_element_type=jnp.float32)
            if res_kind == "identity":
                acc = acc + r_ref[s * TV:(s + 1) * TV, :].astype(jnp.float32)
            acc = acc + bt_ref[...]
            o_ref[s * TV:(s + 1) * TV, :] = jnp.maximum(acc, 0.0).astype(
                o_ref.dtype)

    return _body


def _temporal64(y_rows, res_rows, wg0, wg1, w8, bt, *, SB, T, V,
                out_dtype):
    M, C = y_rows.shape
    rows = SB * T * V
    body = _make_temporal64_body("zero" if res_rows is None else "identity",
                                 SB=SB, T=T, V=V, C=C)
    w_specs = [pl.BlockSpec((4 * C, C), lambda g: (0, 0)),
               pl.BlockSpec((4 * C, C), lambda g: (0, 0)),
               pl.BlockSpec((C, C), lambda g: (0, 0)),
               pl.BlockSpec((1, C), lambda g: (0, 0))]
    if res_rows is None:
        args = (y_rows, wg0, wg1, w8, bt)
        in_specs = [pl.BlockSpec((rows, C), lambda g: (g, 0))] + w_specs
    else:
        args = (y_rows, res_rows, wg0, wg1, w8, bt)
        in_specs = [pl.BlockSpec((rows, C), lambda g: (g, 0)),
                    pl.BlockSpec((rows, C), lambda g: (g, 0))] + w_specs
    return pl.pallas_call(
        body,
        out_shape=jax.ShapeDtypeStruct((M, C), out_dtype),
        grid=(M // rows,),
        in_specs=in_specs,
        out_specs=pl.BlockSpec((rows, C), lambda g: (g, 0)),
        compiler_params=_CP,
    )(*args)


def _make_temporal32_body(*, SB, T, V, C):
    TV = T * V
    PAD = 4 * V

    def _body(y_ref, r_ref, wg_ref, w8_ref, wr_ref, b_ref, o_ref):
        z = jnp.zeros((PAD, C), _BF)
        for s in range(SB):
            ys = y_ref[s * TV:(s + 1) * TV, :]
            yp = jnp.concatenate([z, ys, z], axis=0)
            q = jnp.concatenate([yp[32 * i:32 * i + TV] for i in range(8)],
                                axis=1)
            acc = jnp.dot(q, wg_ref[...],
                          preferred_element_type=jnp.float32)
            acc = acc + jnp.dot(yp[256:TV + 256], w8_ref[...],
                                preferred_element_type=jnp.float32)
            acc = acc + jnp.dot(r_ref[s * TV:(s + 1) * TV, :], wr_ref[...],
                                preferred_element_type=jnp.float32)
            acc = acc + b_ref[...]
            o_ref[s * TV:(s + 1) * TV, :] = jnp.maximum(acc, 0.0).astype(
                o_ref.dtype)

    return _body


def _temporal32(y_rows, res_rows, wg, w8, wr, b, *, SB, T, V):
    M, C = y_rows.shape
    Cr = res_rows.shape[1]
    rows = SB * T * V
    body = _make_temporal32_body(SB=SB, T=T, V=V, C=C)
    return pl.pallas_call(
        body,
        out_shape=jax.ShapeDtypeStruct((M, C), jnp.float32),
        grid=(M // rows,),
        in_specs=[pl.BlockSpec((rows, C), lambda g: (g, 0)),
                  pl.BlockSpec((rows, Cr), lambda g: (g, 0)),
                  pl.BlockSpec((8 * C, C), lambda g: (0, 0)),
                  pl.BlockSpec((C, C), lambda g: (0, 0)),
                  pl.BlockSpec((Cr, C), lambda g: (0, 0)),
                  pl.BlockSpec((1, C), lambda g: (0, 0))],
        out_specs=pl.BlockSpec((rows, C), lambda g: (g, 0)),
        compiler_params=_CP,
    )(y_rows, res_rows, wg, w8, wr, b)


def _pack_taps(wt, lo, hi):
    n = hi - lo
    return wt[lo:hi].reshape(n * wt.shape[1], wt.shape[2]).astype(_BF)


def kernel(x,
           l0_wbig, l0_bsp, l0_wt, l0_bt,
           l1_wbig, l1_bsp, l1_wt, l1_bt,
           l2_wbig, l2_bsp, l2_wt, l2_bt, l2_wres, l2_bres):
    N, C, T, V = x.shape
    SB = 4
    if N % SB:
        pad = SB - N % SB
        x = jnp.pad(x, ((0, pad), (0, 0), (0, 0), (0, 0)))
    Np = x.shape[0]
    rows = SB * T

    x2d = jnp.transpose(x, (0, 2, 3, 1)).reshape(Np * T, V * C).astype(_BF)

    y0 = _spatial(x2d, l0_wbig.astype(_BF), l0_bsp, rows)
    t0 = _temporal64(y0.reshape(Np * T * V, 64), None,
                     _pack_taps(l0_wt, 0, 4), _pack_taps(l0_wt, 4, 8),
                     l0_wt[8].astype(_BF), l0_bt,
                     SB=SB, T=T, V=V, out_dtype=_BF)

    y1 = _spatial(t0.reshape(Np * T, V * 64), l1_wbig.astype(_BF), l1_bsp,
                  rows)
    t1 = _temporal64(y1.reshape(Np * T * V, 64), t0,
                     _pack_taps(l1_wt, 0, 4), _pack_taps(l1_wt, 4, 8),
                     l1_wt[8].astype(_BF), l1_bt,
                     SB=SB, T=T, V=V, out_dtype=_BF)

    y2 = _spatial(t1.reshape(Np * T, V * 64), l2_wbig.astype(_BF), l2_bsp,
                  rows)
    out = _temporal32(y2.reshape(Np * T * V, 32), t1,
                      _pack_taps(l2_wt, 0, 8), l2_wt[8].astype(_BF),
                      l2_wres.astype(_BF), l2_bt + l2_bres,
                      SB=SB, T=T, V=V)

    out = out.reshape(Np, T, V, 32)[:N]
    return jnp.transpose(out, (0, 3, 1, 2))
